# baseline (speedup 1.0000x reference)
import functools

import jax
import jax.numpy as jnp
from jax.experimental import pallas as pl
from jax.experimental.pallas import tpu as pltpu

B = 256
N = 64
ENC = 128
DEC = 64
DEC2 = 32
CB = 4
STEPS = N // CB
K1 = N * DEC
P1 = N * DEC2


def _goggle_kernel(x_ref, we_ref, be_ref, wmu_ref, bmu_ref, wlv_ref, blv_ref,
                   g_ref, gt_ref, w0f_ref, ccf_ref, w1_hbm, b1c_ref, w2m_ref,
                   b2_ref, it_ref, eps_ref,
                   xhat_ref, adj_ref, mu_ref, lv_ref,
                   bz_ref, h1t_ref, adjt_ref, w1buf, sems):
    k = pl.program_id(0)

    def w1_dma(slot, blk):
        return pltpu.make_async_copy(
            w1_hbm.at[:, pl.ds(blk * CB, CB)], w1buf.at[slot], sems.at[slot])

    @pl.when(k == 0)
    def _prologue():
        w1_dma(0, 0).start()

        h = jax.nn.relu(jnp.dot(x_ref[...], we_ref[...],
                                preferred_element_type=jnp.float32) + be_ref[...])
        mu = jnp.dot(h, wmu_ref[...], preferred_element_type=jnp.float32) + bmu_ref[...]
        lv = jnp.dot(h, wlv_ref[...], preferred_element_type=jnp.float32) + blv_ref[...]
        mu_ref[...] = mu
        lv_ref[...] = lv
        z = mu + eps_ref[...] * jnp.exp(0.5 * lv)

        r_id = jax.lax.broadcasted_iota(jnp.int32, (N, N), 0)
        c_id = jax.lax.broadcasted_iota(jnp.int32, (N, N), 1)
        eye = (r_id == c_id).astype(jnp.float32)
        thr = it_ref[0, 0] > 50.0
        adj = jax.nn.sigmoid(g_ref[...]) * (1.0 - eye) + eye
        adj_ref[...] = jnp.where(jnp.logical_and(thr, adj <= 0.1), 0.0, adj)
        adjt = jax.nn.sigmoid(gt_ref[...]) * (1.0 - eye) + eye
        adjt_ref[...] = jnp.where(jnp.logical_and(thr, adjt <= 0.1), 0.0, adjt)

        ez = (jax.lax.broadcasted_iota(jnp.int32, (N, K1), 1) // DEC
              == jax.lax.broadcasted_iota(jnp.int32, (N, K1), 0)).astype(jnp.float32)
        zexp = jnp.dot(z, ez, preferred_element_type=jnp.float32)
        bz_ref[...] = jnp.tanh(zexp * w0f_ref[...] + ccf_ref[...])

    slot = jax.lax.rem(k, 2)
    @pl.when(k + 1 < STEPS)
    def _prefetch():
        w1_dma(jax.lax.rem(k + 1, 2), k + 1).start()
    w1_dma(slot, k).wait()

    adjcols = adjt_ref[pl.ds(k * CB, CB), :]
    er = (jax.lax.broadcasted_iota(jnp.int32, (K1, N), 0) // DEC
          == jax.lax.broadcasted_iota(jnp.int32, (K1, N), 1)).astype(jnp.float32)
    sblk = jax.lax.dot_general(er, adjcols, (((1,), (1,)), ((), ())),
                               preferred_element_type=jnp.float32)

    for j in range(CB):
        wj = w1buf[slot, :, j].reshape(K1, DEC2) * sblk[:, j:j + 1]
        resj = jax.lax.dot_general(wj, bz_ref[...], (((0,), (1,)), ((), ())),
                                   preferred_element_type=jnp.float32)
        h1t_ref[pl.ds((k * CB + j) * DEC2, DEC2), :] = jax.nn.relu(resj + b1c_ref[...])

    @pl.when(k == STEPS - 1)
    def _epilogue():
        er2 = (jax.lax.broadcasted_iota(jnp.int32, (P1, N), 0) // DEC2
               == jax.lax.broadcasted_iota(jnp.int32, (P1, N), 1)).astype(jnp.float32)
        s2 = jnp.dot(er2, adj_ref[...], preferred_element_type=jnp.float32)
        xhatt = jax.lax.dot_general(w2m_ref[...] * s2, h1t_ref[...],
                                    (((0,), (0,)), ((), ())),
                                    preferred_element_type=jnp.float32)
        xhat_ref[...] = xhatt.T + b2_ref[...]


@functools.partial(jax.jit, static_argnames=())
def kernel(x, We, be, Wmu, bmu, Wlv, blv, G, Wemb, bemb, W1, b1, W2, b2, iter):
    f32 = jnp.float32
    w0f = Wemb[:, 0, :].reshape(1, K1)
    ccf = (Wemb[jnp.arange(N), jnp.arange(N) + 1, :] + bemb).reshape(1, K1)
    w2m = W2[:, :, :, 0].transpose(0, 2, 1).reshape(P1, N)
    b1c = b1.reshape(DEC2, 1)
    eps = jax.random.normal(jax.random.key(42), (B, N), dtype=f32)
    it = jnp.asarray(iter, dtype=f32).reshape(1, 1)

    resident = lambda s: pl.BlockSpec(s, lambda k: (0,) * len(s))
    out = pl.pallas_call(
        _goggle_kernel,
        grid=(STEPS,),
        in_specs=[
            resident((B, N)),
            resident((N, ENC)),
            resident((1, ENC)),
            resident((ENC, N)),
            resident((1, N)),
            resident((ENC, N)),
            resident((1, N)),
            resident((N, N)),
            resident((N, N)),
            resident((1, K1)),
            resident((1, K1)),
            pl.BlockSpec(memory_space=pl.ANY),
            resident((DEC2, 1)),
            resident((P1, N)),
            resident((1, 1)),
            resident((1, 1)),
            resident((B, N)),
        ],
        out_specs=(
            resident((B, N)),
            resident((N, N)),
            resident((B, N)),
            resident((B, N)),
        ),
        out_shape=(
            jax.ShapeDtypeStruct((B, N), f32),
            jax.ShapeDtypeStruct((N, N), f32),
            jax.ShapeDtypeStruct((B, N), f32),
            jax.ShapeDtypeStruct((B, N), f32),
        ),
        scratch_shapes=[
            pltpu.VMEM((B, K1), f32),
            pltpu.VMEM((P1, B), f32),
            pltpu.VMEM((N, N), f32),
            pltpu.VMEM((2, N, CB, DEC, DEC2), f32),
            pltpu.SemaphoreType.DMA((2,)),
        ],
        compiler_params=pltpu.CompilerParams(
            dimension_semantics=("arbitrary",),
        ),
    )(x, We, be.reshape(1, ENC), Wmu, bmu.reshape(1, N), Wlv, blv.reshape(1, N),
      G, G.T, w0f, ccf, W1, b1c, w2m, b2.reshape(1, 1), it, eps)
    return out


# speedup vs baseline: 3.0738x; 3.0738x over previous
import functools

import jax
import jax.numpy as jnp
from jax.experimental import pallas as pl
from jax.experimental.pallas import tpu as pltpu

B = 256
N = 64
ENC = 128
DEC = 64
DEC2 = 32
RB = 8
STEPS = N // RB
K1 = N * DEC
P1 = N * DEC2
BF = jnp.bfloat16


def _goggle_kernel(x_ref, we_ref, be_ref, wmu_ref, bmu_ref, wlv_ref, blv_ref,
                   g_ref, w0c_ref, ccc_ref, w1_ref, b1c_ref, w2t_ref, b2_ref,
                   it_ref, eps_ref,
                   xhat_ref, adj_ref, mu_ref, lv_ref,
                   bzt_ref, acc_ref):
    k = pl.program_id(0)

    @pl.when(k == 0)
    def _prologue():
        h = jax.nn.relu(jnp.dot(x_ref[...], we_ref[...],
                                preferred_element_type=jnp.float32) + be_ref[...])
        mu = jnp.dot(h, wmu_ref[...], preferred_element_type=jnp.float32) + bmu_ref[...]
        lv = jnp.dot(h, wlv_ref[...], preferred_element_type=jnp.float32) + blv_ref[...]
        mu_ref[...] = mu
        lv_ref[...] = lv
        z = mu + eps_ref[...] * jnp.exp(0.5 * lv)

        r_id = jax.lax.broadcasted_iota(jnp.int32, (N, N), 0)
        c_id = jax.lax.broadcasted_iota(jnp.int32, (N, N), 1)
        eye = (r_id == c_id).astype(jnp.float32)
        adj = jax.nn.sigmoid(g_ref[...]) * (1.0 - eye) + eye
        adj_ref[...] = jnp.where(
            jnp.logical_and(it_ref[0, 0] > 50.0, adj <= 0.1), 0.0, adj)

        zt = z.T
        ez = (jax.lax.broadcasted_iota(jnp.int32, (K1, N), 0) // DEC
              == jax.lax.broadcasted_iota(jnp.int32, (K1, N), 1)).astype(jnp.float32)
        zexpt = jnp.dot(ez, zt, preferred_element_type=jnp.float32)
        bzt_ref[...] = jnp.tanh(zexpt * w0c_ref[...] + ccc_ref[...]).astype(BF)
        acc_ref[...] = jnp.zeros((P1, B), dtype=jnp.float32)

    adjb = adj_ref[pl.ds(k * RB, RB), :]
    ec = (jax.lax.broadcasted_iota(jnp.int32, (P1, N), 0) // DEC2
          == jax.lax.broadcasted_iota(jnp.int32, (P1, N), 1)).astype(jnp.float32)
    sblk = jax.lax.dot_general(ec, adjb, (((1,), (1,)), ((), ())),
                               preferred_element_type=jnp.float32)

    pieces = [
        (w1_ref[rr].reshape(P1, DEC) * sblk[:, rr:rr + 1]).astype(BF)
        for rr in range(RB)
    ]
    wcat = jnp.concatenate(pieces, axis=1)
    bzb = bzt_ref[pl.ds(k * RB * DEC, RB * DEC), :]
    acc_ref[...] += jnp.dot(wcat, bzb, preferred_element_type=jnp.float32)

    @pl.when(k == STEPS - 1)
    def _epilogue():
        h1t = jax.nn.relu(acc_ref[...] + b1c_ref[...]).astype(BF)
        ec2 = (jax.lax.broadcasted_iota(jnp.int32, (N, P1), 1) // DEC2
               == jax.lax.broadcasted_iota(jnp.int32, (N, P1), 0)).astype(jnp.float32)
        s2 = jnp.dot(adj_ref[...].T, ec2, preferred_element_type=jnp.float32)
        w2s = (w2t_ref[...] * s2).astype(BF)
        xhatt = jnp.dot(w2s, h1t, preferred_element_type=jnp.float32)
        xhat_ref[...] = xhatt.T + b2_ref[...]


@functools.partial(jax.jit, static_argnames=())
def kernel(x, We, be, Wmu, bmu, Wlv, blv, G, Wemb, bemb, W1, b1, W2, b2, iter):
    f32 = jnp.float32
    w1v = W1.transpose(0, 1, 3, 2)
    w0c = Wemb[:, 0, :].reshape(K1, 1)
    ccc = (Wemb[jnp.arange(N), jnp.arange(N) + 1, :] + bemb).reshape(K1, 1)
    w2t = W2[:, :, :, 0].transpose(1, 0, 2).reshape(N, P1)
    b1c = jnp.tile(b1, N).reshape(P1, 1)
    eps = jax.random.normal(jax.random.key(42), (B, N), dtype=f32)
    it = jnp.asarray(iter, dtype=f32).reshape(1, 1)

    resident = lambda s: pl.BlockSpec(s, lambda k: (0,) * len(s))
    out = pl.pallas_call(
        _goggle_kernel,
        grid=(STEPS,),
        in_specs=[
            resident((B, N)),
            resident((N, ENC)),
            resident((1, ENC)),
            resident((ENC, N)),
            resident((1, N)),
            resident((ENC, N)),
            resident((1, N)),
            resident((N, N)),
            resident((K1, 1)),
            resident((K1, 1)),
            pl.BlockSpec((RB, N, DEC2, DEC), lambda k: (k, 0, 0, 0)),
            resident((P1, 1)),
            resident((N, P1)),
            resident((1, 1)),
            resident((1, 1)),
            resident((B, N)),
        ],
        out_specs=(
            resident((B, N)),
            resident((N, N)),
            resident((B, N)),
            resident((B, N)),
        ),
        out_shape=(
            jax.ShapeDtypeStruct((B, N), f32),
            jax.ShapeDtypeStruct((N, N), f32),
            jax.ShapeDtypeStruct((B, N), f32),
            jax.ShapeDtypeStruct((B, N), f32),
        ),
        scratch_shapes=[
            pltpu.VMEM((K1, B), BF),
            pltpu.VMEM((P1, B), f32),
        ],
        compiler_params=pltpu.CompilerParams(
            dimension_semantics=("arbitrary",),
        ),
    )(x, We, be.reshape(1, ENC), Wmu, bmu.reshape(1, N), Wlv, blv.reshape(1, N),
      G, w0c, ccc, w1v, b1c, w2t, b2.reshape(1, 1), it, eps)
    return out


# speedup vs baseline: 3.3388x; 1.0862x over previous
import functools

import jax
import jax.numpy as jnp
from jax.experimental import pallas as pl
from jax.experimental.pallas import tpu as pltpu

B = 256
N = 64
ENC = 128
DEC = 64
DEC2 = 32
RB = 8
STEPS = N // RB
K1 = N * DEC
P1 = N * DEC2
BF = jnp.bfloat16


def _goggle_kernel(xt_ref, we_ref, be_ref, wmut_ref, bmu_ref, wlvt_ref, blv_ref,
                   g_ref, w0c_ref, ccc_ref, w1_ref, b1c_ref, w2t_ref, b2_ref,
                   it_ref, eps_ref,
                   xhat_ref, adj_ref, mu_ref, lv_ref,
                   bzt_ref, acc_ref):
    k = pl.program_id(0)

    @pl.when(k == 0)
    def _prologue():
        h = jax.nn.relu(jnp.dot(xt_ref[...].T, we_ref[...],
                                preferred_element_type=jnp.float32) + be_ref[...])
        mu = jax.lax.dot_general(h, wmut_ref[...], (((1,), (1,)), ((), ())),
                                 preferred_element_type=jnp.float32) + bmu_ref[...]
        lv = jax.lax.dot_general(h, wlvt_ref[...], (((1,), (1,)), ((), ())),
                                 preferred_element_type=jnp.float32) + blv_ref[...]
        mu_ref[...] = mu
        lv_ref[...] = lv
        z = mu + eps_ref[...] * jnp.exp(0.5 * lv)

        r_id = jax.lax.broadcasted_iota(jnp.int32, (N, N), 0)
        c_id = jax.lax.broadcasted_iota(jnp.int32, (N, N), 1)
        eye = (r_id == c_id).astype(jnp.float32)
        adj = jax.nn.sigmoid(g_ref[...]) * (1.0 - eye) + eye
        adj_ref[...] = jnp.where(
            jnp.logical_and(it_ref[0, 0] > 50.0, adj <= 0.1), 0.0, adj)

        zt = z.T
        ez = (jax.lax.broadcasted_iota(jnp.int32, (K1, N), 0) // DEC
              == jax.lax.broadcasted_iota(jnp.int32, (K1, N), 1)).astype(jnp.float32)
        zexpt = jnp.dot(ez, zt, preferred_element_type=jnp.float32)
        bzt_ref[...] = jnp.tanh(zexpt * w0c_ref[...] + ccc_ref[...]).astype(BF)
        acc_ref[...] = jnp.zeros((P1, B), dtype=jnp.float32)

    adjb = adj_ref[pl.ds(k * RB, RB), :]
    ec = (jax.lax.broadcasted_iota(jnp.int32, (P1, N), 0) // DEC2
          == jax.lax.broadcasted_iota(jnp.int32, (P1, N), 1)).astype(jnp.float32)
    sblk = jax.lax.dot_general(ec, adjb, (((1,), (1,)), ((), ())),
                               preferred_element_type=jnp.float32)

    pieces = [
        (w1_ref[rr].reshape(P1, DEC) * sblk[:, rr:rr + 1]).astype(BF)
        for rr in range(RB)
    ]
    wcat = jnp.concatenate(pieces, axis=1)
    bzb = bzt_ref[pl.ds(k * RB * DEC, RB * DEC), :]
    acc_ref[...] += jnp.dot(wcat, bzb, preferred_element_type=jnp.float32)

    @pl.when(k == STEPS - 1)
    def _epilogue():
        h1t = jax.nn.relu(acc_ref[...] + b1c_ref[...]).astype(BF)
        ec2 = (jax.lax.broadcasted_iota(jnp.int32, (N, P1), 1) // DEC2
               == jax.lax.broadcasted_iota(jnp.int32, (N, P1), 0)).astype(jnp.float32)
        s2 = jnp.dot(adj_ref[...].T, ec2, preferred_element_type=jnp.float32)
        w2s = (w2t_ref[...] * s2).astype(BF)
        xhatt = jnp.dot(w2s, h1t, preferred_element_type=jnp.float32)
        xhat_ref[...] = xhatt.T + b2_ref[...]


@functools.partial(jax.jit, static_argnames=())
def kernel(x, We, be, Wmu, bmu, Wlv, blv, G, Wemb, bemb, W1, b1, W2, b2, iter):
    f32 = jnp.float32
    w1v = W1.transpose(0, 1, 3, 2)
    w0c = Wemb[:, 0, :].reshape(K1, 1)
    ccc = (Wemb[jnp.arange(N), jnp.arange(N) + 1, :] + bemb).reshape(K1, 1)
    w2t = W2[:, :, :, 0].transpose(1, 0, 2).reshape(N, P1)
    b1c = jnp.tile(b1, N).reshape(P1, 1)
    eps = jax.random.normal(jax.random.key(42), (B, N), dtype=f32)
    it = jnp.asarray(iter, dtype=f32).reshape(1, 1)

    resident = lambda s: pl.BlockSpec(s, lambda k: (0,) * len(s))
    out = pl.pallas_call(
        _goggle_kernel,
        grid=(STEPS,),
        in_specs=[
            resident((N, B)),
            resident((N, ENC)),
            resident((1, ENC)),
            resident((N, ENC)),
            resident((1, N)),
            resident((N, ENC)),
            resident((1, N)),
            resident((N, N)),
            resident((K1, 1)),
            resident((K1, 1)),
            pl.BlockSpec((RB, N, DEC2, DEC), lambda k: (k, 0, 0, 0)),
            resident((P1, 1)),
            resident((N, P1)),
            resident((1, 1)),
            resident((1, 1)),
            resident((B, N)),
        ],
        out_specs=(
            resident((B, N)),
            resident((N, N)),
            resident((B, N)),
            resident((B, N)),
        ),
        out_shape=(
            jax.ShapeDtypeStruct((B, N), f32),
            jax.ShapeDtypeStruct((N, N), f32),
            jax.ShapeDtypeStruct((B, N), f32),
            jax.ShapeDtypeStruct((B, N), f32),
        ),
        scratch_shapes=[
            pltpu.VMEM((K1, B), BF),
            pltpu.VMEM((P1, B), f32),
        ],
        compiler_params=pltpu.CompilerParams(
            dimension_semantics=("arbitrary",),
        ),
    )(x.T, We, be.reshape(1, ENC), Wmu.T, bmu.reshape(1, N), Wlv.T,
      blv.reshape(1, N), G, w0c, ccc, w1v, b1c, w2t, b2.reshape(1, 1), it, eps)
    return out


# speedup vs baseline: 3.6392x; 1.0900x over previous
import functools

import jax
import jax.numpy as jnp
from jax.experimental import pallas as pl
from jax.experimental.pallas import tpu as pltpu

B = 256
N = 64
ENC = 128
DEC = 64
DEC2 = 32
RB = 8
STEPS = N // RB
K1 = N * DEC
P1 = N * DEC2
BF = jnp.bfloat16


def _goggle_kernel(xt_ref, we_ref, be_ref, wmut_ref, bmu_ref, wlvt_ref, blv_ref,
                   g_ref, w0c_ref, ccc_ref, w1_ref, b1c_ref, w2t_ref, b2_ref,
                   it_ref, eps_ref,
                   xhatt_ref, adj_ref, mut_ref, lvt_ref,
                   bzt_ref, acc_ref):
    k = pl.program_id(0)

    @pl.when(k == 0)
    def _prologue():
        h = jax.nn.relu(jnp.dot(xt_ref[...].T, we_ref[...],
                                preferred_element_type=jnp.float32) + be_ref[...])
        mu = jax.lax.dot_general(h, wmut_ref[...], (((1,), (1,)), ((), ())),
                                 preferred_element_type=jnp.float32) + bmu_ref[...]
        lv = jax.lax.dot_general(h, wlvt_ref[...], (((1,), (1,)), ((), ())),
                                 preferred_element_type=jnp.float32) + blv_ref[...]
        mut_ref[...] = mu.T
        lvt_ref[...] = lv.T
        z = mu + eps_ref[...] * jnp.exp(0.5 * lv)

        r_id = jax.lax.broadcasted_iota(jnp.int32, (N, N), 0)
        c_id = jax.lax.broadcasted_iota(jnp.int32, (N, N), 1)
        eye = (r_id == c_id).astype(jnp.float32)
        adj = jax.nn.sigmoid(g_ref[...]) * (1.0 - eye) + eye
        adj_ref[...] = jnp.where(
            jnp.logical_and(it_ref[0, 0] > 50.0, adj <= 0.1), 0.0, adj)

        zt = z.T
        ez = (jax.lax.broadcasted_iota(jnp.int32, (K1, N), 0) // DEC
              == jax.lax.broadcasted_iota(jnp.int32, (K1, N), 1)).astype(jnp.float32)
        zexpt = jnp.dot(ez, zt, preferred_element_type=jnp.float32)
        bzt_ref[...] = jnp.tanh(zexpt * w0c_ref[...] + ccc_ref[...]).astype(BF)
        acc_ref[...] = jnp.zeros((P1, B), dtype=jnp.float32)

    adjb = adj_ref[pl.ds(k * RB, RB), :]
    ec = (jax.lax.broadcasted_iota(jnp.int32, (P1, N), 0) // DEC2
          == jax.lax.broadcasted_iota(jnp.int32, (P1, N), 1)).astype(jnp.float32)
    sblk = jax.lax.dot_general(ec, adjb, (((1,), (1,)), ((), ())),
                               preferred_element_type=jnp.float32)

    pieces = [
        (w1_ref[rr].reshape(P1, DEC) * sblk[:, rr:rr + 1]).astype(BF)
        for rr in range(RB)
    ]
    wcat = jnp.concatenate(pieces, axis=1)
    bzb = bzt_ref[pl.ds(k * RB * DEC, RB * DEC), :]
    acc_ref[...] += jnp.dot(wcat, bzb, preferred_element_type=jnp.float32)

    @pl.when(k == STEPS - 1)
    def _epilogue():
        h1t = jax.nn.relu(acc_ref[...] + b1c_ref[...]).astype(BF)
        ec2 = (jax.lax.broadcasted_iota(jnp.int32, (N, P1), 1) // DEC2
               == jax.lax.broadcasted_iota(jnp.int32, (N, P1), 0)).astype(jnp.float32)
        s2 = jnp.dot(adj_ref[...].T, ec2, preferred_element_type=jnp.float32)
        w2s = (w2t_ref[...] * s2).astype(BF)
        xhatt = jnp.dot(w2s, h1t, preferred_element_type=jnp.float32)
        xhatt_ref[...] = xhatt + b2_ref[...]


@functools.partial(jax.jit, static_argnames=())
def kernel(x, We, be, Wmu, bmu, Wlv, blv, G, Wemb, bemb, W1, b1, W2, b2, iter):
    f32 = jnp.float32
    w1v = W1.transpose(0, 1, 3, 2)
    w0c = Wemb[:, 0, :].reshape(K1, 1)
    ccc = (Wemb[jnp.arange(N), jnp.arange(N) + 1, :] + bemb).reshape(K1, 1)
    w2t = W2[:, :, :, 0].transpose(1, 0, 2).reshape(N, P1)
    b1c = jnp.tile(b1, N).reshape(P1, 1)
    eps = jax.random.normal(jax.random.key(42), (B, N), dtype=f32)
    it = jnp.asarray(iter, dtype=f32).reshape(1, 1)

    resident = lambda s: pl.BlockSpec(s, lambda k: (0,) * len(s))
    out = pl.pallas_call(
        _goggle_kernel,
        grid=(STEPS,),
        in_specs=[
            resident((N, B)),
            resident((N, ENC)),
            resident((1, ENC)),
            resident((N, ENC)),
            resident((1, N)),
            resident((N, ENC)),
            resident((1, N)),
            resident((N, N)),
            resident((K1, 1)),
            resident((K1, 1)),
            pl.BlockSpec((RB, N, DEC2, DEC), lambda k: (k, 0, 0, 0)),
            resident((P1, 1)),
            resident((N, P1)),
            resident((1, 1)),
            resident((1, 1)),
            resident((B, N)),
        ],
        out_specs=(
            resident((N, B)),
            resident((N, N)),
            resident((N, B)),
            resident((N, B)),
        ),
        out_shape=(
            jax.ShapeDtypeStruct((N, B), f32),
            jax.ShapeDtypeStruct((N, N), f32),
            jax.ShapeDtypeStruct((N, B), f32),
            jax.ShapeDtypeStruct((N, B), f32),
        ),
        scratch_shapes=[
            pltpu.VMEM((K1, B), BF),
            pltpu.VMEM((P1, B), f32),
        ],
        compiler_params=pltpu.CompilerParams(
            dimension_semantics=("arbitrary",),
        ),
    )(x.T, We, be.reshape(1, ENC), Wmu.T, bmu.reshape(1, N), Wlv.T,
      blv.reshape(1, N), G, w0c, ccc, w1v, b1c, w2t, b2.reshape(1, 1), it, eps)
    return (out[0].T, out[1], out[2].T, out[3].T)


# speedup vs baseline: 4.0501x; 1.1129x over previous
import functools

import jax
import jax.numpy as jnp
from jax.experimental import pallas as pl
from jax.experimental.pallas import tpu as pltpu

B = 256
N = 64
ENC = 128
DEC = 64
DEC2 = 32
RB = 8
STEPS = N // RB
K1 = N * DEC
P1 = N * DEC2
BF = jnp.bfloat16


def _goggle_kernel(xt_ref, we_ref, be_ref, wmut_ref, bmu_ref, wlvt_ref, blv_ref,
                   g_ref, w0c_ref, ccc_ref, w1_ref, b1c_ref, w2t_ref, b2_ref,
                   it_ref, eps_ref,
                   xhatt_ref, adj_ref, mut_ref, lvt_ref,
                   bzt_ref, acc_ref):
    k = pl.program_id(0)

    @pl.when(k == 0)
    def _prologue():
        h = jax.nn.relu(jnp.dot(xt_ref[...].T, we_ref[...],
                                preferred_element_type=jnp.float32) + be_ref[...])
        mu = jax.lax.dot_general(h, wmut_ref[...], (((1,), (1,)), ((), ())),
                                 preferred_element_type=jnp.float32) + bmu_ref[...]
        lv = jax.lax.dot_general(h, wlvt_ref[...], (((1,), (1,)), ((), ())),
                                 preferred_element_type=jnp.float32) + blv_ref[...]
        mut_ref[...] = mu.T
        lvt_ref[...] = lv.T
        z = mu + eps_ref[...] * jnp.exp(0.5 * lv)

        r_id = jax.lax.broadcasted_iota(jnp.int32, (N, N), 0)
        c_id = jax.lax.broadcasted_iota(jnp.int32, (N, N), 1)
        eye = (r_id == c_id).astype(jnp.float32)
        adj = jax.nn.sigmoid(g_ref[...]) * (1.0 - eye) + eye
        adj_ref[...] = jnp.where(
            jnp.logical_and(it_ref[0, 0] > 50.0, adj <= 0.1), 0.0, adj)

        zt = z.T
        ez = (jax.lax.broadcasted_iota(jnp.int32, (K1, N), 0) // DEC
              == jax.lax.broadcasted_iota(jnp.int32, (K1, N), 1)).astype(jnp.float32)
        zexpt = jnp.dot(ez, zt, preferred_element_type=jnp.float32)
        bzt_ref[...] = jnp.tanh(zexpt * w0c_ref[...] + ccc_ref[...]).astype(BF)
        acc_ref[...] = jnp.zeros((P1, B), dtype=jnp.float32)

    adjb = adj_ref[pl.ds(k * RB, RB), :]
    ec = (jax.lax.broadcasted_iota(jnp.int32, (P1, N), 0) // DEC2
          == jax.lax.broadcasted_iota(jnp.int32, (P1, N), 1)).astype(jnp.float32)
    sblk = jax.lax.dot_general(ec, adjb, (((1,), (1,)), ((), ())),
                               preferred_element_type=jnp.float32)

    sblk_bf = sblk.astype(BF)
    pieces = [
        w1_ref[rr].reshape(P1, DEC).astype(BF) * sblk_bf[:, rr:rr + 1]
        for rr in range(RB)
    ]
    wcat = jnp.concatenate(pieces, axis=1)
    bzb = bzt_ref[pl.ds(k * RB * DEC, RB * DEC), :]
    acc_ref[...] += jnp.dot(wcat, bzb, preferred_element_type=jnp.float32)

    @pl.when(k == STEPS - 1)
    def _epilogue():
        h1t = jax.nn.relu(acc_ref[...] + b1c_ref[...]).astype(BF)
        ec2 = (jax.lax.broadcasted_iota(jnp.int32, (N, P1), 1) // DEC2
               == jax.lax.broadcasted_iota(jnp.int32, (N, P1), 0)).astype(jnp.float32)
        s2 = jnp.dot(adj_ref[...].T, ec2, preferred_element_type=jnp.float32)
        w2s = (w2t_ref[...] * s2).astype(BF)
        xhatt = jnp.dot(w2s, h1t, preferred_element_type=jnp.float32)
        xhatt_ref[...] = xhatt + b2_ref[...]


@functools.partial(jax.jit, static_argnames=())
def kernel(x, We, be, Wmu, bmu, Wlv, blv, G, Wemb, bemb, W1, b1, W2, b2, iter):
    f32 = jnp.float32
    w1v = W1.transpose(0, 1, 3, 2)
    w0c = Wemb[:, 0, :].reshape(K1, 1)
    ccc = (Wemb[jnp.arange(N), jnp.arange(N) + 1, :] + bemb).reshape(K1, 1)
    w2t = W2[:, :, :, 0].transpose(1, 0, 2).reshape(N, P1)
    b1c = jnp.tile(b1, N).reshape(P1, 1)
    eps = jax.random.normal(jax.random.key(42), (B, N), dtype=f32)
    it = jnp.asarray(iter, dtype=f32).reshape(1, 1)

    resident = lambda s: pl.BlockSpec(s, lambda k: (0,) * len(s))
    out = pl.pallas_call(
        _goggle_kernel,
        grid=(STEPS,),
        in_specs=[
            resident((N, B)),
            resident((N, ENC)),
            resident((1, ENC)),
            resident((N, ENC)),
            resident((1, N)),
            resident((N, ENC)),
            resident((1, N)),
            resident((N, N)),
            resident((K1, 1)),
            resident((K1, 1)),
            pl.BlockSpec((RB, N, DEC2, DEC), lambda k: (k, 0, 0, 0)),
            resident((P1, 1)),
            resident((N, P1)),
            resident((1, 1)),
            resident((1, 1)),
            resident((B, N)),
        ],
        out_specs=(
            resident((N, B)),
            resident((N, N)),
            resident((N, B)),
            resident((N, B)),
        ),
        out_shape=(
            jax.ShapeDtypeStruct((N, B), f32),
            jax.ShapeDtypeStruct((N, N), f32),
            jax.ShapeDtypeStruct((N, B), f32),
            jax.ShapeDtypeStruct((N, B), f32),
        ),
        scratch_shapes=[
            pltpu.VMEM((K1, B), BF),
            pltpu.VMEM((P1, B), f32),
        ],
        compiler_params=pltpu.CompilerParams(
            dimension_semantics=("arbitrary",),
        ),
    )(x.T, We, be.reshape(1, ENC), Wmu.T, bmu.reshape(1, N), Wlv.T,
      blv.reshape(1, N), G, w0c, ccc, w1v, b1c, w2t, b2.reshape(1, 1), it, eps)
    return (out[0].T, out[1], out[2].T, out[3].T)


# speedup vs baseline: 4.0689x; 1.0046x over previous
import functools

import jax
import jax.numpy as jnp
from jax.experimental import pallas as pl
from jax.experimental.pallas import tpu as pltpu

B = 256
N = 64
ENC = 128
DEC = 64
DEC2 = 32
RB = 8
STEPS = N // RB
K1 = N * DEC
P1 = N * DEC2
BF = jnp.bfloat16


def _goggle_kernel(xt_ref, we_ref, be_ref, wmut_ref, bmu_ref, wlvt_ref, blv_ref,
                   g_ref, w0c_ref, ccc_ref, w1_ref, b1c_ref, w2t_ref, b2_ref,
                   it_ref, eps_ref,
                   xhatt_ref, adj_ref, mut_ref, lvt_ref,
                   bzt_ref, acc_ref):
    k = pl.program_id(0)

    @pl.when(k == 0)
    def _prologue():
        h = jax.nn.relu(jnp.dot(xt_ref[...].T, we_ref[...],
                                preferred_element_type=jnp.float32) + be_ref[...])
        mu = jax.lax.dot_general(h, wmut_ref[...], (((1,), (1,)), ((), ())),
                                 preferred_element_type=jnp.float32) + bmu_ref[...]
        lv = jax.lax.dot_general(h, wlvt_ref[...], (((1,), (1,)), ((), ())),
                                 preferred_element_type=jnp.float32) + blv_ref[...]
        mut_ref[...] = mu.T
        lvt_ref[...] = lv.T
        z = mu + eps_ref[...] * jnp.exp(0.5 * lv)

        r_id = jax.lax.broadcasted_iota(jnp.int32, (N, N), 0)
        c_id = jax.lax.broadcasted_iota(jnp.int32, (N, N), 1)
        eye = (r_id == c_id).astype(jnp.float32)
        adj = jax.nn.sigmoid(g_ref[...]) * (1.0 - eye) + eye
        adj_ref[...] = jnp.where(
            jnp.logical_and(it_ref[0, 0] > 50.0, adj <= 0.1), 0.0, adj)

        zt = z.T
        ez = (jax.lax.broadcasted_iota(jnp.int32, (K1, N), 0) // DEC
              == jax.lax.broadcasted_iota(jnp.int32, (K1, N), 1)).astype(jnp.float32)
        zexpt = jnp.dot(ez, zt, preferred_element_type=jnp.float32)
        bzt_ref[...] = jnp.tanh(zexpt * w0c_ref[...] + ccc_ref[...]).astype(BF)
        acc_ref[...] = jnp.zeros((P1, B), dtype=jnp.float32)

    adjb = adj_ref[pl.ds(k * RB, RB), :].astype(BF)
    ec = (jax.lax.broadcasted_iota(jnp.int32, (P1, N), 0) // DEC2
          == jax.lax.broadcasted_iota(jnp.int32, (P1, N), 1)).astype(BF)
    sblk = jax.lax.dot_general(ec, adjb, (((1,), (1,)), ((), ())),
                               preferred_element_type=jnp.float32)

    sblk_bf = sblk.astype(BF)
    pieces = [
        w1_ref[rr].reshape(P1, DEC).astype(BF) * sblk_bf[:, rr:rr + 1]
        for rr in range(RB)
    ]
    wcat = jnp.concatenate(pieces, axis=1)
    bzb = bzt_ref[pl.ds(k * RB * DEC, RB * DEC), :]
    acc_ref[...] += jnp.dot(wcat, bzb, preferred_element_type=jnp.float32)

    @pl.when(k == STEPS - 1)
    def _epilogue():
        h1t = jax.nn.relu(acc_ref[...] + b1c_ref[...]).astype(BF)
        ec2 = (jax.lax.broadcasted_iota(jnp.int32, (N, P1), 1) // DEC2
               == jax.lax.broadcasted_iota(jnp.int32, (N, P1), 0)).astype(jnp.float32)
        s2 = jnp.dot(adj_ref[...].T, ec2, preferred_element_type=jnp.float32)
        w2s = (w2t_ref[...] * s2).astype(BF)
        xhatt = jnp.dot(w2s, h1t, preferred_element_type=jnp.float32)
        xhatt_ref[...] = xhatt + b2_ref[...]


@functools.partial(jax.jit, static_argnames=())
def kernel(x, We, be, Wmu, bmu, Wlv, blv, G, Wemb, bemb, W1, b1, W2, b2, iter):
    f32 = jnp.float32
    w1v = W1.transpose(0, 1, 3, 2)
    w0c = Wemb[:, 0, :].reshape(K1, 1)
    ccc = (Wemb[jnp.arange(N), jnp.arange(N) + 1, :] + bemb).reshape(K1, 1)
    w2t = W2[:, :, :, 0].transpose(1, 0, 2).reshape(N, P1)
    b1c = jnp.tile(b1, N).reshape(P1, 1)
    eps = jax.random.normal(jax.random.key(42), (B, N), dtype=f32)
    it = jnp.asarray(iter, dtype=f32).reshape(1, 1)

    resident = lambda s: pl.BlockSpec(s, lambda k: (0,) * len(s))
    out = pl.pallas_call(
        _goggle_kernel,
        grid=(STEPS,),
        in_specs=[
            resident((N, B)),
            resident((N, ENC)),
            resident((1, ENC)),
            resident((N, ENC)),
            resident((1, N)),
            resident((N, ENC)),
            resident((1, N)),
            resident((N, N)),
            resident((K1, 1)),
            resident((K1, 1)),
            pl.BlockSpec((RB, N, DEC2, DEC), lambda k: (k, 0, 0, 0)),
            resident((P1, 1)),
            resident((N, P1)),
            resident((1, 1)),
            resident((1, 1)),
            resident((B, N)),
        ],
        out_specs=(
            resident((N, B)),
            resident((N, N)),
            resident((N, B)),
            resident((N, B)),
        ),
        out_shape=(
            jax.ShapeDtypeStruct((N, B), f32),
            jax.ShapeDtypeStruct((N, N), f32),
            jax.ShapeDtypeStruct((N, B), f32),
            jax.ShapeDtypeStruct((N, B), f32),
        ),
        scratch_shapes=[
            pltpu.VMEM((K1, B), BF),
            pltpu.VMEM((P1, B), f32),
        ],
        compiler_params=pltpu.CompilerParams(
            dimension_semantics=("arbitrary",),
        ),
    )(x.T, We, be.reshape(1, ENC), Wmu.T, bmu.reshape(1, N), Wlv.T,
      blv.reshape(1, N), G, w0c, ccc, w1v, b1c, w2t, b2.reshape(1, 1), it, eps)
    return (out[0].T, out[1], out[2].T, out[3].T)
